# baseline (speedup 1.0000x reference)
import functools

import jax
import jax.numpy as jnp
from jax import lax
from jax.experimental import pallas as pl
from jax.experimental.pallas import tpu as pltpu
from jax.experimental.pallas import tpu_sc as plsc

_N = 65536
_R, _C = 512, 128
_KAPPA = 0.85
_NGRID = 24

_NC, _NS = 2, 16
_NW = _NC * _NS
_CH = _N // _NW


def _tc_pre_body(c_ref, a_ref, s_ref, key_ref, rho_ref, w_ref):
    row = lax.broadcasted_iota(jnp.int32, (_R, _C), 0)
    lane = lax.broadcasted_iota(jnp.int32, (_R, _C), 1)
    imat = row * _C + lane

    sex = s_ref[...]
    cent = c_ref[...]
    age = a_ref[...]

    acc = jnp.zeros((_R, _C), jnp.float32)
    inv_k = jnp.float32(1.0 / _KAPPA)
    for j in range(_NGRID):
        t = (age - jnp.float32(0.25 * j)) * inv_k
        acc = acc + jnp.exp(jnp.float32(-0.5) * t * t)
    w_ref[...] = acc

    x = sex
    for sh in (1, 2, 4, 8, 16, 32, 64):
        x = x + jnp.where(lane >= sh, pltpu.roll(x, sh, 1), 0)
    rowsum = jnp.sum(sex, axis=1, keepdims=True)
    rb = jnp.broadcast_to(rowsum, (_R, _C))
    for sh in (1, 2, 4, 8, 16, 32, 64, 128, 256):
        rb = rb + jnp.where(row >= sh, pltpu.roll(rb, sh, 0), 0)
    cum1 = x + (rb - rowsum)
    n1 = jnp.sum(sex)
    n0 = _N - n1
    rho_ref[...] = jnp.where(sex == 1, n0 + cum1 - 1, imat - cum1)

    key_ref[...] = cent + jnp.float32(2.0) * sex.astype(jnp.float32)


_tc_pre = pl.pallas_call(
    _tc_pre_body,
    out_shape=[
        jax.ShapeDtypeStruct((_R, _C), jnp.float32),
        jax.ShapeDtypeStruct((_R, _C), jnp.int32),
        jax.ShapeDtypeStruct((_R, _C), jnp.float32),
    ],
)


def _tc_sort_body(key_ref, g_ref):
    row = lax.broadcasted_iota(jnp.int32, (_R, _C), 0)
    lane = lax.broadcasted_iota(jnp.int32, (_R, _C), 1)
    imat = row * _C + lane

    xk = key_ref[...]
    n1 = jnp.sum((xk >= jnp.float32(2.0)).astype(jnp.int32))
    n0 = _N - n1

    for k in range(1, 17):
        asc = (imat & (1 << k)) == 0
        for lj in range(k - 1, -1, -1):
            j = 1 << lj
            if j >= _C:
                jr = j // _C
                x4 = xk.reshape(_R // (2 * jr), 2, jr, _C)
                partner = jnp.concatenate([x4[:, 1:], x4[:, :1]], 1)
                partner = partner.reshape(_R, _C)
                bit = (row & jr) != 0
            else:
                up = pltpu.roll(xk, j, 1)
                dn = pltpu.roll(xk, _C - j, 1)
                bit = (lane & j) != 0
                partner = jnp.where(bit, up, dn)
            mn = jnp.minimum(xk, partner)
            mx = jnp.maximum(xk, partner)
            take_min = (~bit) == asc
            xk = jnp.where(take_min, mn, mx)

    grp1 = imat >= n0
    val = xk - jnp.where(grp1, jnp.float32(2.0), jnp.float32(0.0))
    ng = jnp.where(grp1, n1, n0)
    rank = jnp.where(grp1, imat - n0, imat)
    start = jnp.float32(0.01)
    stop = jnp.float32(0.99)
    denom = jnp.maximum(ng - 1, 1).astype(jnp.float32)
    delta = (stop - start) / denom
    u = start + rank.astype(jnp.float32) * delta
    u = jnp.where((rank == ng - 1) & (ng > 1), stop, u)
    g_ref[...] = jnp.abs(val - u)


_tc_sort = pl.pallas_call(
    _tc_sort_body,
    out_shape=jax.ShapeDtypeStruct((_R, _C), jnp.float32),
)


def _tc_dot_body(g_ref, wp_ref, out_ref):
    prod = g_ref[...] * wp_ref[...]
    out_ref[...] = jnp.sum(prod.reshape(64, 8, _C), axis=0)


_tc_dot = pl.pallas_call(
    _tc_dot_body,
    out_shape=jax.ShapeDtypeStruct((8, _C), jnp.float32),
)


def _sc_body(rho_hbm, w_hbm, wp_hbm, idx_v, w_v, sem):
    wid = lax.axis_index("s") * _NC + lax.axis_index("c")
    base = wid * _CH
    pltpu.sync_copy(rho_hbm.at[pl.ds(base, _CH)], idx_v)
    pltpu.sync_copy(w_hbm.at[pl.ds(base, _CH)], w_v)
    pltpu.async_copy(w_v, wp_hbm.at[idx_v], sem).wait()


@functools.cache
def _sc_scatter():
    return pl.kernel(
        _sc_body,
        out_type=jax.ShapeDtypeStruct((_N,), jnp.float32),
        mesh=plsc.VectorSubcoreMesh(
            core_axis_name="c", subcore_axis_name="s", num_cores=_NC, num_subcores=_NS
        ),
        compiler_params=pltpu.CompilerParams(needs_layout_passes=False),
        scratch_types=[
            pltpu.VMEM((_CH,), jnp.int32),
            pltpu.VMEM((_CH,), jnp.float32),
            pltpu.SemaphoreType.DMA,
        ],
    )


def kernel(centiles, ages, sexes, datasets):
    del datasets
    key, rho, w = _tc_pre(
        centiles.reshape(_R, _C), ages.reshape(_R, _C), sexes.reshape(_R, _C)
    )
    wp = _sc_scatter()(rho.reshape(_N), w.reshape(_N))
    g = _tc_sort(key)
    partials = _tc_dot(g, wp.reshape(_R, _C))
    return jnp.sum(partials) * jnp.float32(1.0 / _N)


# speedup vs baseline: 4.3279x; 4.3279x over previous
import functools

import jax
import jax.numpy as jnp
from jax import lax
from jax.experimental import pallas as pl
from jax.experimental.pallas import tpu as pltpu
from jax.experimental.pallas import tpu_sc as plsc

_N = 65536
_R, _C = 512, 128
_KAPPA = 0.85
_NGRID = 24

_NC, _NS = 2, 16
_NW = _NC * _NS
_CH = _N // _NW


def _tc_pre_body(c_ref, a_ref, s_ref, key_ref, rho_ref, w_ref):
    row = lax.broadcasted_iota(jnp.int32, (_R, _C), 0)
    lane = lax.broadcasted_iota(jnp.int32, (_R, _C), 1)
    imat = row * _C + lane

    sex = s_ref[...]
    cent = c_ref[...]
    age = a_ref[...]

    acc = jnp.zeros((_R, _C), jnp.float32)
    inv_k = jnp.float32(1.0 / _KAPPA)
    for j in range(_NGRID):
        t = (age - jnp.float32(0.25 * j)) * inv_k
        acc = acc + jnp.exp(jnp.float32(-0.5) * t * t)
    w_ref[...] = acc

    x = sex
    for sh in (1, 2, 4, 8, 16, 32, 64):
        x = x + jnp.where(lane >= sh, pltpu.roll(x, sh, 1), 0)
    rowsum = jnp.sum(sex, axis=1, keepdims=True)
    rb = jnp.broadcast_to(rowsum, (_R, _C))
    for sh in (1, 2, 4, 8, 16, 32, 64, 128, 256):
        rb = rb + jnp.where(row >= sh, pltpu.roll(rb, sh, 0), 0)
    cum1 = x + (rb - rowsum)
    n1 = jnp.sum(sex)
    n0 = _N - n1
    rho_ref[...] = jnp.where(sex == 1, n0 + cum1 - 1, imat - cum1)

    key_ref[...] = cent + jnp.float32(2.0) * sex.astype(jnp.float32)


_tc_pre = pl.pallas_call(
    _tc_pre_body,
    out_shape=[
        jax.ShapeDtypeStruct((_R, _C), jnp.float32),
        jax.ShapeDtypeStruct((_R, _C), jnp.int32),
        jax.ShapeDtypeStruct((_R, _C), jnp.float32),
    ],
)


def _tc_sort_body(key_ref, g_ref):
    row = lax.broadcasted_iota(jnp.int32, (_R, _C), 0)
    lane = lax.broadcasted_iota(jnp.int32, (_R, _C), 1)
    imat = row * _C + lane

    xk = key_ref[...]
    n1 = jnp.sum((xk >= jnp.float32(2.0)).astype(jnp.int32))
    n0 = _N - n1

    for k in range(1, 17):
        asc = (imat & (1 << k)) == 0
        for lj in range(k - 1, -1, -1):
            j = 1 << lj
            if j >= _C:
                jr = j // _C
                x4 = xk.reshape(_R // (2 * jr), 2, jr, _C)
                partner = jnp.concatenate([x4[:, 1:], x4[:, :1]], 1)
                partner = partner.reshape(_R, _C)
                bit = (row & jr) != 0
            else:
                up = pltpu.roll(xk, j, 1)
                dn = pltpu.roll(xk, _C - j, 1)
                bit = (lane & j) != 0
                partner = jnp.where(bit, up, dn)
            mn = jnp.minimum(xk, partner)
            mx = jnp.maximum(xk, partner)
            take_min = (~bit) == asc
            xk = jnp.where(take_min, mn, mx)

    grp1 = imat >= n0
    val = xk - jnp.where(grp1, jnp.float32(2.0), jnp.float32(0.0))
    ng = jnp.where(grp1, n1, n0)
    rank = jnp.where(grp1, imat - n0, imat)
    start = jnp.float32(0.01)
    stop = jnp.float32(0.99)
    denom = jnp.maximum(ng - 1, 1).astype(jnp.float32)
    delta = (stop - start) / denom
    u = start + rank.astype(jnp.float32) * delta
    u = jnp.where((rank == ng - 1) & (ng > 1), stop, u)
    g_ref[...] = jnp.abs(val - u)


_tc_sort = pl.pallas_call(
    _tc_sort_body,
    out_shape=jax.ShapeDtypeStruct((_R, _C), jnp.float32),
)


def _tc_dot_body(g_ref, wp0_ref, wp1_ref, out_ref):
    prod = g_ref[...] * (wp0_ref[...] + wp1_ref[...])
    out_ref[...] = jnp.sum(prod.reshape(64, 8, _C), axis=0)


_tc_dot = pl.pallas_call(
    _tc_dot_body,
    out_shape=jax.ShapeDtypeStruct((8, _C), jnp.float32),
)

_SL = _N // _NS


def _sc_body(rho_hbm, w_hbm, wp_hbm, idx_v, w_v, z_v, wp_sh):
    cid = lax.axis_index("c")
    sid = lax.axis_index("s")
    wid = sid * _NC + cid
    base = wid * _CH

    def zbody(i, _):
        z_v[pl.ds(i * 16, 16)] = jnp.zeros((16,), jnp.float32)
        return 0

    lax.fori_loop(0, _SL // 16, zbody, 0)
    pltpu.sync_copy(z_v, wp_sh.at[pl.ds(sid * _SL, _SL)])
    pltpu.sync_copy(rho_hbm.at[pl.ds(base, _CH)], idx_v)
    pltpu.sync_copy(w_hbm.at[pl.ds(base, _CH)], w_v)
    plsc.subcore_barrier()
    pltpu.sync_copy(w_v, wp_sh.at[idx_v], add=True)
    plsc.subcore_barrier()
    pltpu.sync_copy(wp_sh.at[pl.ds(sid * _SL, _SL)], wp_hbm.at[cid, pl.ds(sid * _SL, _SL)])


@functools.cache
def _sc_scatter():
    return pl.kernel(
        _sc_body,
        out_type=jax.ShapeDtypeStruct((_NC, _N), jnp.float32),
        mesh=plsc.VectorSubcoreMesh(
            core_axis_name="c", subcore_axis_name="s", num_cores=_NC, num_subcores=_NS
        ),
        compiler_params=pltpu.CompilerParams(needs_layout_passes=False),
        scratch_types=[
            pltpu.VMEM((_CH,), jnp.int32),
            pltpu.VMEM((_CH,), jnp.float32),
            pltpu.VMEM((_SL,), jnp.float32),
            pltpu.VMEM_SHARED((_N,), jnp.float32),
        ],
    )


def kernel(centiles, ages, sexes, datasets):
    del datasets
    key, rho, w = _tc_pre(
        centiles.reshape(_R, _C), ages.reshape(_R, _C), sexes.reshape(_R, _C)
    )
    wp = _sc_scatter()(rho.reshape(_N), w.reshape(_N))
    g = _tc_sort(key)
    partials = _tc_dot(g, wp[0].reshape(_R, _C), wp[1].reshape(_R, _C))
    return jnp.sum(partials) * jnp.float32(1.0 / _N)


# speedup vs baseline: 4.5359x; 1.0480x over previous
import functools

import jax
import jax.numpy as jnp
from jax import lax
from jax.experimental import pallas as pl
from jax.experimental.pallas import tpu as pltpu
from jax.experimental.pallas import tpu_sc as plsc

_N = 65536
_R, _C = 512, 128
_KAPPA = 0.85
_NGRID = 24

_NC, _NS = 2, 16
_NW = _NC * _NS
_CH = _N // _NW


def _tc_pre_body(c_ref, a_ref, s_ref, key_ref, rho_ref, w_ref):
    row = lax.broadcasted_iota(jnp.int32, (_R, _C), 0)
    lane = lax.broadcasted_iota(jnp.int32, (_R, _C), 1)
    imat = row * _C + lane

    sex = s_ref[...]
    cent = c_ref[...]
    age = a_ref[...]

    acc = jnp.zeros((_R, _C), jnp.float32)
    inv_k = jnp.float32(1.0 / _KAPPA)
    for j in range(_NGRID):
        t = (age - jnp.float32(0.25 * j)) * inv_k
        acc = acc + jnp.exp(jnp.float32(-0.5) * t * t)
    w_ref[...] = acc

    x = sex
    for sh in (1, 2, 4, 8, 16, 32, 64):
        x = x + jnp.where(lane >= sh, pltpu.roll(x, sh, 1), 0)
    rowsum = jnp.sum(sex, axis=1, keepdims=True)
    rb = jnp.broadcast_to(rowsum, (_R, _C))
    for sh in (1, 2, 4, 8, 16, 32, 64, 128, 256):
        rb = rb + jnp.where(row >= sh, pltpu.roll(rb, sh, 0), 0)
    cum1 = x + (rb - rowsum)
    n1 = jnp.sum(sex)
    n0 = _N - n1
    rho = jnp.where(sex == 1, n0 + cum1 - 1, imat - cum1)
    rho_ref[...] = ((rho & (_R - 1)) << 7) | (rho >> 9)

    key_ref[...] = cent + jnp.float32(2.0) * sex.astype(jnp.float32)


_tc_pre = pl.pallas_call(
    _tc_pre_body,
    out_shape=[
        jax.ShapeDtypeStruct((_R, _C), jnp.float32),
        jax.ShapeDtypeStruct((_R, _C), jnp.int32),
        jax.ShapeDtypeStruct((_R, _C), jnp.float32),
    ],
)


def _tc_sort_body(key_ref, g_ref):
    row = lax.broadcasted_iota(jnp.int32, (_R, _C), 0)
    lane = lax.broadcasted_iota(jnp.int32, (_R, _C), 1)
    imat = lane * _R + row

    xk = key_ref[...]
    n1 = jnp.sum((xk >= jnp.float32(2.0)).astype(jnp.int32))
    n0 = _N - n1

    for k in range(1, 17):
        asc = (imat & (1 << k)) == 0
        for lj in range(k - 1, -1, -1):
            j = 1 << lj
            if j >= _R:
                jl = j // _R
                up = pltpu.roll(xk, jl, 1)
                dn = pltpu.roll(xk, _C - jl, 1)
                bit = (lane & jl) != 0
                partner = jnp.where(bit, up, dn)
            else:
                x4 = xk.reshape(_R // (2 * j), 2, j, _C)
                partner = jnp.concatenate([x4[:, 1:], x4[:, :1]], 1)
                partner = partner.reshape(_R, _C)
                bit = (row & j) != 0
            mn = jnp.minimum(xk, partner)
            mx = jnp.maximum(xk, partner)
            take_min = (~bit) == asc
            xk = jnp.where(take_min, mn, mx)

    grp1 = imat >= n0
    val = xk - jnp.where(grp1, jnp.float32(2.0), jnp.float32(0.0))
    ng = jnp.where(grp1, n1, n0)
    rank = jnp.where(grp1, imat - n0, imat)
    start = jnp.float32(0.01)
    stop = jnp.float32(0.99)
    denom = jnp.maximum(ng - 1, 1).astype(jnp.float32)
    delta = (stop - start) / denom
    u = start + rank.astype(jnp.float32) * delta
    u = jnp.where((rank == ng - 1) & (ng > 1), stop, u)
    g_ref[...] = jnp.abs(val - u)


_tc_sort = pl.pallas_call(
    _tc_sort_body,
    out_shape=jax.ShapeDtypeStruct((_R, _C), jnp.float32),
)


def _tc_dot_body(g_ref, wp0_ref, wp1_ref, out_ref):
    prod = g_ref[...] * (wp0_ref[...] + wp1_ref[...])
    out_ref[...] = jnp.sum(prod.reshape(64, 8, _C), axis=0)


_tc_dot = pl.pallas_call(
    _tc_dot_body,
    out_shape=jax.ShapeDtypeStruct((8, _C), jnp.float32),
)

_SL = _N // _NS


def _sc_body(rho_hbm, w_hbm, wp_hbm, idx_v, w_v, z_v, wp_sh):
    cid = lax.axis_index("c")
    sid = lax.axis_index("s")
    wid = sid * _NC + cid
    base = wid * _CH

    def zbody(i, _):
        z_v[pl.ds(i * 16, 16)] = jnp.zeros((16,), jnp.float32)
        return 0

    lax.fori_loop(0, _SL // 16, zbody, 0)
    pltpu.sync_copy(z_v, wp_sh.at[pl.ds(sid * _SL, _SL)])
    pltpu.sync_copy(rho_hbm.at[pl.ds(base, _CH)], idx_v)
    pltpu.sync_copy(w_hbm.at[pl.ds(base, _CH)], w_v)
    plsc.subcore_barrier()
    pltpu.sync_copy(w_v, wp_sh.at[idx_v], add=True)
    plsc.subcore_barrier()
    pltpu.sync_copy(wp_sh.at[pl.ds(sid * _SL, _SL)], wp_hbm.at[cid, pl.ds(sid * _SL, _SL)])


@functools.cache
def _sc_scatter():
    return pl.kernel(
        _sc_body,
        out_type=jax.ShapeDtypeStruct((_NC, _N), jnp.float32),
        mesh=plsc.VectorSubcoreMesh(
            core_axis_name="c", subcore_axis_name="s", num_cores=_NC, num_subcores=_NS
        ),
        compiler_params=pltpu.CompilerParams(needs_layout_passes=False),
        scratch_types=[
            pltpu.VMEM((_CH,), jnp.int32),
            pltpu.VMEM((_CH,), jnp.float32),
            pltpu.VMEM((_SL,), jnp.float32),
            pltpu.VMEM_SHARED((_N,), jnp.float32),
        ],
    )


def kernel(centiles, ages, sexes, datasets):
    del datasets
    key, rho, w = _tc_pre(
        centiles.reshape(_R, _C), ages.reshape(_R, _C), sexes.reshape(_R, _C)
    )
    wp = _sc_scatter()(rho.reshape(_N), w.reshape(_N))
    g = _tc_sort(key)
    partials = _tc_dot(g, wp[0].reshape(_R, _C), wp[1].reshape(_R, _C))
    return jnp.sum(partials) * jnp.float32(1.0 / _N)


# speedup vs baseline: 4.8340x; 1.0657x over previous
import functools

import jax
import jax.numpy as jnp
from jax import lax
from jax.experimental import pallas as pl
from jax.experimental.pallas import tpu as pltpu
from jax.experimental.pallas import tpu_sc as plsc

_N = 65536
_R, _C = 512, 128
_KAPPA = 0.85
_NGRID = 24

_NC, _NS = 2, 16
_NW = _NC * _NS
_CH = _N // _NW


def _tc_pre_body(a_ref, s_ref, rho_ref, w_ref):
    row = lax.broadcasted_iota(jnp.int32, (_R, _C), 0)
    lane = lax.broadcasted_iota(jnp.int32, (_R, _C), 1)
    imat = row * _C + lane

    sex = s_ref[...]
    age = a_ref[...]

    acc = jnp.zeros((_R, _C), jnp.float32)
    inv_k = jnp.float32(1.0 / _KAPPA)
    for j in range(_NGRID):
        t = (age - jnp.float32(0.25 * j)) * inv_k
        acc = acc + jnp.exp(jnp.float32(-0.5) * t * t)
    w_ref[...] = acc

    x = sex
    for sh in (1, 2, 4, 8, 16, 32, 64):
        x = x + jnp.where(lane >= sh, pltpu.roll(x, sh, 1), 0)
    rowsum = jnp.sum(sex, axis=1, keepdims=True)
    rb = jnp.broadcast_to(rowsum, (_R, _C))
    for sh in (1, 2, 4, 8, 16, 32, 64, 128, 256):
        rb = rb + jnp.where(row >= sh, pltpu.roll(rb, sh, 0), 0)
    cum1 = x + (rb - rowsum)
    n1 = jnp.sum(sex)
    n0 = _N - n1
    rho = jnp.where(sex == 1, n0 + cum1 - 1, imat - cum1)
    rho_ref[...] = ((rho & (_R - 1)) << 7) | (rho >> 9)


_tc_pre = pl.pallas_call(
    _tc_pre_body,
    out_shape=[
        jax.ShapeDtypeStruct((_R, _C), jnp.int32),
        jax.ShapeDtypeStruct((_R, _C), jnp.float32),
    ],
)


def _tc_sort_body(c_ref, s_ref, g_ref):
    row = lax.broadcasted_iota(jnp.int32, (_R, _C), 0)
    lane = lax.broadcasted_iota(jnp.int32, (_R, _C), 1)
    imat = lane * _R + row

    sex = s_ref[...]
    xk = c_ref[...] + jnp.float32(2.0) * sex.astype(jnp.float32)
    n1 = jnp.sum(sex)
    n0 = _N - n1

    for k in range(1, 17):
        asc = (imat & (1 << k)) == 0
        for lj in range(k - 1, -1, -1):
            j = 1 << lj
            if j >= _R:
                jl = j // _R
                up = pltpu.roll(xk, jl, 1)
                dn = pltpu.roll(xk, _C - jl, 1)
                bit = (lane & jl) != 0
                partner = jnp.where(bit, up, dn)
            else:
                x4 = xk.reshape(_R // (2 * j), 2, j, _C)
                partner = jnp.concatenate([x4[:, 1:], x4[:, :1]], 1)
                partner = partner.reshape(_R, _C)
                bit = (row & j) != 0
            mn = jnp.minimum(xk, partner)
            mx = jnp.maximum(xk, partner)
            take_min = (~bit) == asc
            xk = jnp.where(take_min, mn, mx)

    grp1 = imat >= n0
    val = xk - jnp.where(grp1, jnp.float32(2.0), jnp.float32(0.0))
    ng = jnp.where(grp1, n1, n0)
    rank = jnp.where(grp1, imat - n0, imat)
    start = jnp.float32(0.01)
    stop = jnp.float32(0.99)
    denom = jnp.maximum(ng - 1, 1).astype(jnp.float32)
    delta = (stop - start) / denom
    u = start + rank.astype(jnp.float32) * delta
    u = jnp.where((rank == ng - 1) & (ng > 1), stop, u)
    g_ref[...] = jnp.abs(val - u)


_tc_sort = pl.pallas_call(
    _tc_sort_body,
    out_shape=jax.ShapeDtypeStruct((_R, _C), jnp.float32),
)


def _tc_dot_body(g_ref, wp0_ref, wp1_ref, out_ref):
    prod = g_ref[...] * (wp0_ref[...] + wp1_ref[...])
    out_ref[...] = (jnp.sum(prod) * jnp.float32(1.0 / _N)).reshape(1, 1)


_tc_dot = pl.pallas_call(
    _tc_dot_body,
    out_shape=jax.ShapeDtypeStruct((1, 1), jnp.float32),
)

_SL = _N // _NS


def _sc_body(rho_hbm, w_hbm, wp_hbm, idx_v, w_v, z_v, wp_sh):
    cid = lax.axis_index("c")
    sid = lax.axis_index("s")
    wid = sid * _NC + cid
    base = wid * _CH

    def zbody(i, _):
        z_v[pl.ds(i * 16, 16)] = jnp.zeros((16,), jnp.float32)
        return 0

    lax.fori_loop(0, _SL // 16, zbody, 0)
    pltpu.sync_copy(z_v, wp_sh.at[pl.ds(sid * _SL, _SL)])
    pltpu.sync_copy(rho_hbm.at[pl.ds(base, _CH)], idx_v)
    pltpu.sync_copy(w_hbm.at[pl.ds(base, _CH)], w_v)
    plsc.subcore_barrier()
    pltpu.sync_copy(w_v, wp_sh.at[idx_v], add=True)
    plsc.subcore_barrier()
    pltpu.sync_copy(wp_sh.at[pl.ds(sid * _SL, _SL)], wp_hbm.at[cid, pl.ds(sid * _SL, _SL)])


@functools.cache
def _sc_scatter():
    return pl.kernel(
        _sc_body,
        out_type=jax.ShapeDtypeStruct((_NC, _N), jnp.float32),
        mesh=plsc.VectorSubcoreMesh(
            core_axis_name="c", subcore_axis_name="s", num_cores=_NC, num_subcores=_NS
        ),
        compiler_params=pltpu.CompilerParams(needs_layout_passes=False),
        scratch_types=[
            pltpu.VMEM((_CH,), jnp.int32),
            pltpu.VMEM((_CH,), jnp.float32),
            pltpu.VMEM((_SL,), jnp.float32),
            pltpu.VMEM_SHARED((_N,), jnp.float32),
        ],
    )


def kernel(centiles, ages, sexes, datasets):
    del datasets
    rho, w = _tc_pre(ages.reshape(_R, _C), sexes.reshape(_R, _C))
    wp = _sc_scatter()(rho.reshape(_N), w.reshape(_N))
    g = _tc_sort(centiles.reshape(_R, _C), sexes.reshape(_R, _C))
    partials = _tc_dot(g, wp[0].reshape(_R, _C), wp[1].reshape(_R, _C))
    return partials[0, 0]
